# Initial kernel scaffold; baseline (speedup 1.0000x reference)
import jax
import jax.numpy as jnp
from jax.experimental import pallas as pl


def kernel(xyz, features, sample_inds, sa_w1, sa_g1, sa_b1, sa_w2, sa_g2, sa_b2, sa_w3, sa_g3, sa_b3, c1_w, c1_b, bn1_g, bn1_b, c2_w, c2_b, bn2_g, bn2_b, gat_W, gat_a, gat_Wo, gat_ao, c3_w, c3_b):
    raise NotImplementedError("write your pallas kernel here")


# speedup vs baseline: 6.0608x; 6.0608x over previous
import functools
import math

import jax
import jax.numpy as jnp
from jax.experimental import pallas as pl
from jax.experimental.pallas import tpu as pltpu
from jax.experimental.pallas import tpu_sc as plsc

B, N, C = 8, 1024, 256
S = 256
NSAMPLE = 16
RADIUS = 0.3
NHEADS = 4
NHID = 128
OUT_CH = 119
INV_BN = 1.0 / math.sqrt(1.0 + 1e-5)
HIGH = jax.lax.Precision.HIGHEST


def _sc_gather_rows(x, idx, window):
    n = idx.shape[0]
    V = x.shape[1]
    idx2 = idx.reshape(1, n)
    mesh = plsc.VectorSubcoreMesh(core_axis_name="core", subcore_axis_name="subcore")

    @pl.kernel(out_type=jax.ShapeDtypeStruct((n, V), x.dtype), mesh=mesh)
    def k(x_hbm, i_hbm, o_hbm):
        def body(i_vmem, o_vmem):
            pltpu.sync_copy(x_hbm.at[i_vmem.at[0]], o_vmem)

        pltpu.emit_pipeline(
            body,
            grid=(n // window,),
            in_specs=[pl.BlockSpec((1, window), lambda i: (0, i))],
            out_specs=[pl.BlockSpec((window, V), lambda i: (i, 0))],
            core_axis_name=("core", "subcore"),
            dimension_semantics=(pltpu.PARALLEL,),
        )(i_hbm, o_hbm)

    return k(x, idx2)


def _kernel_a(xyzp_ref, xyzT_ref, featT_ref, cpad_ref, cT_ref, w1xT_ref, w1fT_ref,
              gi_ref, adj_ref, p_ref, q_ref):
    b = pl.program_id(0)
    r2 = RADIUS * RADIUS
    inv_r = 1.0 / RADIUS

    xyzp = xyzp_ref[0]
    featT = featT_ref[0]
    cpad = cpad_ref[0]
    p = (jnp.dot(xyzp * inv_r, w1xT_ref[...], precision=HIGH,
                 preferred_element_type=jnp.float32)
         + jnp.dot(featT, w1fT_ref[...], precision=HIGH,
                   preferred_element_type=jnp.float32))
    p_ref[0] = p
    q_ref[0] = jnp.dot(cpad * inv_r, w1xT_ref[...], precision=HIGH,
                       preferred_element_type=jnp.float32)

    cx = cpad[:, 0:1]
    cy = cpad[:, 1:2]
    cz = cpad[:, 2:3]
    xx = xyzT_ref[0, 0:1, :]
    xy = xyzT_ref[0, 1:2, :]
    xz = xyzT_ref[0, 2:3, :]
    dx = cx - xx
    dy = cy - xy
    dz = cz - xz
    d2 = dx * dx + dy * dy + dz * dz
    iota_n = jax.lax.broadcasted_iota(jnp.int32, (S, N), 1).astype(jnp.float32)
    cand = jnp.where(d2 < r2, iota_n, float(N))
    cols = []
    for _ in range(NSAMPLE):
        m = jnp.min(cand, axis=1, keepdims=True)
        cols.append(m)
        cand = jnp.where(cand == m, float(N), cand)
    gi = jnp.concatenate(cols, axis=1)
    gi = jnp.where(gi == float(N), cols[0], gi)
    gi_ref[0] = gi.astype(jnp.int32) + b * N

    ctx = cT_ref[0, 0:1, :]
    cty = cT_ref[0, 1:2, :]
    ctz = cT_ref[0, 2:3, :]
    ddx = cx - ctx
    ddy = cy - cty
    ddz = cz - ctz
    dist = jnp.sqrt(ddx * ddx + ddy * ddy + ddz * ddz)
    iota_s = jax.lax.broadcasted_iota(jnp.int32, (S, S), 1).astype(jnp.float32)
    amask = jnp.zeros((S, S), jnp.float32)
    for _ in range(NSAMPLE):
        m = jnp.min(dist, axis=1, keepdims=True)
        eq = dist == m
        jsel = jnp.min(jnp.where(eq, iota_s, 4096.0), axis=1, keepdims=True)
        one = iota_s == jsel
        amask = jnp.where(one, 1.0, amask)
        dist = jnp.where(one, 1e30, dist)
    amask = jnp.maximum(amask, jnp.transpose(amask))
    eye = (iota_s ==
           jax.lax.broadcasted_iota(jnp.int32, (S, S), 0).astype(jnp.float32))
    adj_ref[0] = jnp.where(eye, 1.0, amask)


def _gat_head(x, W, a_row, adjmask):
    fp = W.shape[1]
    wh = jnp.dot(x, W, precision=HIGH, preferred_element_type=jnp.float32)
    u = jnp.sum(wh * a_row[:, :fp], axis=1, keepdims=True)
    v = jnp.sum(wh * a_row[:, fp:], axis=1, keepdims=True)
    vT = jnp.transpose(jnp.broadcast_to(v, (S, S)))
    e = u + vT
    e = jnp.where(e > 0, e, 0.2 * e)
    e = jnp.where(adjmask > 0, e, -9e15)
    m = jnp.max(e, axis=1, keepdims=True)
    pexp = jnp.exp(e - m)
    att = pexp / jnp.sum(pexp, axis=1, keepdims=True)
    return jnp.dot(att, wh, precision=HIGH, preferred_element_type=jnp.float32)


def _elu(x):
    return jnp.where(x > 0, x, jnp.exp(x) - 1.0)


def _kernel_b(pg_ref, q_ref, adj_ref, cpad_ref,
              g1_ref, b1_ref, w2T_ref, g2_ref, b2_ref, w3T_ref, g3_ref, b3_ref,
              c1T_ref, c1b_ref, bn1g_ref, bn1b_ref, c2T_ref, c2b_ref, bn2g_ref,
              bn2b_ref, gatW_ref, gata_ref, gatWo_ref, gatao_ref, c3T_ref,
              c3b_ref, out_ref, ctr_ref):
    pg = pg_ref[0].reshape(S, NSAMPLE, 128)
    q = q_ref[0].reshape(S, 1, 128)
    h = pg - q
    h = jnp.maximum(g1_ref[...] * INV_BN * h + b1_ref[...], 0.0)
    h = h.reshape(S * NSAMPLE, 128)
    h = jnp.dot(h, w2T_ref[...], precision=HIGH, preferred_element_type=jnp.float32)
    h = jnp.maximum(g2_ref[...] * INV_BN * h + b2_ref[...], 0.0)
    h = jnp.dot(h, w3T_ref[...], precision=HIGH, preferred_element_type=jnp.float32)
    h = jnp.maximum(g3_ref[...] * INV_BN * h + b3_ref[...], 0.0)
    feat = jnp.max(h.reshape(S, NSAMPLE, 128), axis=1)

    net = jnp.dot(feat, c1T_ref[...], precision=HIGH,
                  preferred_element_type=jnp.float32) + c1b_ref[...]
    net = jnp.maximum(bn1g_ref[...] * net * INV_BN + bn1b_ref[...], 0.0)
    net = jnp.dot(net, c2T_ref[...], precision=HIGH,
                  preferred_element_type=jnp.float32) + c2b_ref[...]
    net = jnp.maximum(bn2g_ref[...] * net * INV_BN + bn2b_ref[...], 0.0)

    adj = adj_ref[0]
    heads = []
    for i in range(NHEADS):
        heads.append(_elu(_gat_head(net, gatW_ref[i], gata_ref[i:i + 1], adj)))
    hcat = jnp.concatenate(heads, axis=1)
    gat_out = _elu(_gat_head(hcat, gatWo_ref[...], gatao_ref[...], adj))

    net3 = jnp.dot(gat_out, c3T_ref[...], precision=HIGH,
                   preferred_element_type=jnp.float32) + c3b_ref[...]
    out_ref[0] = net3
    center = cpad_ref[0][:, 0:3] + net3[:, 2:5]
    ctr_ref[0] = jnp.concatenate([center, jnp.zeros((S, 5), jnp.float32)], axis=1)


def _full(x):
    return pl.BlockSpec(x.shape, lambda b: (0,) * x.ndim)


def _batched(shape):
    nd = len(shape)
    return pl.BlockSpec((1,) + shape, lambda b, _nd=nd: (b,) + (0,) * _nd)


def kernel(xyz, features, sample_inds, sa_w1, sa_g1, sa_b1, sa_w2, sa_g2, sa_b2,
           sa_w3, sa_g3, sa_b3, c1_w, c1_b, bn1_g, bn1_b, c2_w, c2_b, bn2_g,
           bn2_b, gat_W, gat_a, gat_Wo, gat_ao, c3_w, c3_b):
    f32 = jnp.float32
    xyz = xyz.astype(f32)
    xyz_pad = jnp.concatenate([xyz, jnp.zeros((B, N, 13), f32)], axis=2)

    inds_flat = (sample_inds.astype(jnp.int32)
                 + (jnp.arange(B, dtype=jnp.int32) * N)[:, None]).reshape(-1)
    inds_pad = jnp.concatenate([inds_flat,
                                jnp.zeros((B * S,), jnp.int32)])
    xyz_pad128 = jnp.concatenate([xyz, jnp.zeros((B, N, 125), f32)], axis=2)
    cpad = _sc_gather_rows(xyz_pad128.reshape(B * N, 128), inds_pad, 128)
    cpad = cpad[:B * S, :16].reshape(B, S, 16)

    xyzT = jnp.swapaxes(xyz_pad[:, :, :8], 1, 2)
    cT = jnp.swapaxes(cpad[:, :, :8], 1, 2)
    featT = jnp.swapaxes(features.astype(f32), 1, 2)
    w1xT = jnp.concatenate([jnp.transpose(sa_w1[:, :3]).astype(f32),
                            jnp.zeros((13, 128), f32)], axis=0)
    w1fT = jnp.transpose(sa_w1[:, 3:]).astype(f32)

    gi, adjmask, p, q = pl.pallas_call(
        _kernel_a,
        grid=(B,),
        in_specs=[_batched((N, 16)), _batched((8, N)), _batched((N, C)),
                  _batched((S, 16)), _batched((8, S)), _full(w1xT), _full(w1fT)],
        out_specs=[_batched((S, NSAMPLE)), _batched((S, S)), _batched((N, 128)),
                   _batched((S, 128))],
        out_shape=[jax.ShapeDtypeStruct((B, S, NSAMPLE), jnp.int32),
                   jax.ShapeDtypeStruct((B, S, S), f32),
                   jax.ShapeDtypeStruct((B, N, 128), f32),
                   jax.ShapeDtypeStruct((B, S, 128), f32)],
    )(xyz_pad, xyzT, featT, cpad, cT, w1xT, w1fT)

    pg = _sc_gather_rows(p.reshape(B * N, 128), gi.reshape(-1), 128)
    pg = pg.reshape(B, S * NSAMPLE, 128)

    r1 = lambda a: a.astype(f32).reshape(1, -1)
    tr = lambda a: jnp.transpose(a.astype(f32))
    c3T = jnp.concatenate([tr(c3_w), jnp.zeros((128, 128 - OUT_CH), f32)], axis=1)
    c3b = jnp.concatenate([c3_b.astype(f32),
                           jnp.zeros((128 - OUT_CH,), f32)]).reshape(1, -1)

    net3, ctr = pl.pallas_call(
        _kernel_b,
        grid=(B,),
        in_specs=[_batched((S * NSAMPLE, 128)), _batched((S, 128)),
                  _batched((S, S)), _batched((S, 16)),
                  _full(r1(sa_g1)), _full(r1(sa_b1)), _full(tr(sa_w2)),
                  _full(r1(sa_g2)), _full(r1(sa_b2)), _full(tr(sa_w3)),
                  _full(r1(sa_g3)), _full(r1(sa_b3)), _full(tr(c1_w)),
                  _full(r1(c1_b)), _full(r1(bn1_g)), _full(r1(bn1_b)),
                  _full(tr(c2_w)), _full(r1(c2_b)), _full(r1(bn2_g)),
                  _full(r1(bn2_b)), _full(gat_W.astype(f32)),
                  _full(gat_a.astype(f32)), _full(gat_Wo.astype(f32)),
                  _full(gat_ao.astype(f32).reshape(1, -1)), _full(c3T),
                  _full(c3b)],
        out_specs=[_batched((S, 128)), _batched((S, 8))],
        out_shape=[jax.ShapeDtypeStruct((B, S, 128), f32),
                   jax.ShapeDtypeStruct((B, S, 8), f32)],
    )(pg, q, adjmask, cpad, r1(sa_g1), r1(sa_b1), tr(sa_w2), r1(sa_g2),
      r1(sa_b2), tr(sa_w3), r1(sa_g3), r1(sa_b3), tr(c1_w), r1(c1_b),
      r1(bn1_g), r1(bn1_b), tr(c2_w), r1(c2_b), r1(bn2_g), r1(bn2_b),
      gat_W.astype(f32), gat_a.astype(f32), gat_Wo.astype(f32),
      gat_ao.astype(f32).reshape(1, -1), c3T, c3b)

    return jnp.concatenate([net3[:, :, :OUT_CH], ctr[:, :, :3]], axis=2)


# speedup vs baseline: 7.0500x; 1.1632x over previous
import functools
import math

import jax
import jax.numpy as jnp
from jax.experimental import pallas as pl
from jax.experimental.pallas import tpu as pltpu
from jax.experimental.pallas import tpu_sc as plsc

B, N, C = 8, 1024, 256
S = 256
NSAMPLE = 16
RADIUS = 0.3
NHEADS = 4
NHID = 128
OUT_CH = 119
INV_BN = 1.0 / math.sqrt(1.0 + 1e-5)
HIGH = jax.lax.Precision.HIGHEST


def _sc_gather_rows(x, idx, window):
    n = idx.shape[0]
    V = x.shape[1]
    idx2 = idx.reshape(1, n)
    mesh = plsc.VectorSubcoreMesh(core_axis_name="core", subcore_axis_name="subcore")

    @pl.kernel(out_type=jax.ShapeDtypeStruct((n, V), x.dtype), mesh=mesh)
    def k(x_hbm, i_hbm, o_hbm):
        def body(i_vmem, o_vmem):
            pltpu.sync_copy(x_hbm.at[i_vmem.at[0]], o_vmem)

        pltpu.emit_pipeline(
            body,
            grid=(n // window,),
            in_specs=[pl.BlockSpec((1, window), lambda i: (0, i))],
            out_specs=[pl.BlockSpec((window, V), lambda i: (i, 0))],
            core_axis_name=("core", "subcore"),
            dimension_semantics=(pltpu.PARALLEL,),
        )(i_hbm, o_hbm)

    return k(x, idx2)


def _kernel_a(xyzp_ref, xyzT_ref, featT_ref, cpad_ref, cT_ref, w1xT_ref, w1fT_ref,
              gi_ref, adj_ref, p_ref, q_ref):
    b = pl.program_id(0)
    r2 = RADIUS * RADIUS
    inv_r = 1.0 / RADIUS

    xyzp = xyzp_ref[0]
    featT = featT_ref[0]
    cpad = cpad_ref[0]
    p = (jnp.dot(xyzp * inv_r, w1xT_ref[...], precision=HIGH,
                 preferred_element_type=jnp.float32)
         + jnp.dot(featT, w1fT_ref[...], precision=HIGH,
                   preferred_element_type=jnp.float32))
    p_ref[0] = p
    q_ref[0] = jnp.dot(cpad * inv_r, w1xT_ref[...], precision=HIGH,
                       preferred_element_type=jnp.float32)

    cx = cpad[:, 0:1]
    cy = cpad[:, 1:2]
    cz = cpad[:, 2:3]
    xx = xyzT_ref[0, 0:1, :]
    xy = xyzT_ref[0, 1:2, :]
    xz = xyzT_ref[0, 2:3, :]
    dx = cx - xx
    dy = cy - xy
    dz = cz - xz
    d2 = dx * dx + dy * dy + dz * dz
    iota_n = jax.lax.broadcasted_iota(jnp.int32, (S, N), 1).astype(jnp.float32)
    cand = jnp.where(d2 < r2, iota_n, float(N))
    cols = []
    for _ in range(NSAMPLE):
        m = jnp.min(cand, axis=1, keepdims=True)
        cols.append(m)
        cand = jnp.where(cand == m, float(N), cand)
    gi = jnp.concatenate(cols, axis=1)
    gi = jnp.where(gi == float(N), cols[0], gi)
    gi_ref[0] = gi.astype(jnp.int32) + b * N

    ctx = cT_ref[0, 0:1, :]
    cty = cT_ref[0, 1:2, :]
    ctz = cT_ref[0, 2:3, :]
    ddx = cx - ctx
    ddy = cy - cty
    ddz = cz - ctz
    dist = jnp.sqrt(ddx * ddx + ddy * ddy + ddz * ddz)
    iota_s = jax.lax.broadcasted_iota(jnp.int32, (S, S), 1).astype(jnp.float32)
    amask = jnp.zeros((S, S), jnp.float32)
    for _ in range(NSAMPLE):
        m = jnp.min(dist, axis=1, keepdims=True)
        eq = dist == m
        jsel = jnp.min(jnp.where(eq, iota_s, 4096.0), axis=1, keepdims=True)
        one = iota_s == jsel
        amask = jnp.where(one, 1.0, amask)
        dist = jnp.where(one, 1e30, dist)
    amask = jnp.maximum(amask, jnp.transpose(amask))
    eye = (iota_s ==
           jax.lax.broadcasted_iota(jnp.int32, (S, S), 0).astype(jnp.float32))
    adj_ref[0] = jnp.where(eye, 1.0, amask)


def _bdot(a, b):
    return jnp.dot(a.astype(jnp.bfloat16), b.astype(jnp.bfloat16),
                   preferred_element_type=jnp.float32)


def _gat_head(x, W, a_row, adjmask):
    fp = W.shape[1]
    wh = jnp.dot(x, W, precision=HIGH, preferred_element_type=jnp.float32)
    u = jnp.sum(wh * a_row[:, :fp], axis=1, keepdims=True)
    v = jnp.sum(wh * a_row[:, fp:], axis=1, keepdims=True)
    vT = jnp.transpose(jnp.broadcast_to(v, (S, S)))
    e = u + vT
    e = jnp.where(e > 0, e, 0.2 * e)
    e = jnp.where(adjmask > 0, e, -9e15)
    m = jnp.max(e, axis=1, keepdims=True)
    pexp = jnp.exp(e - m)
    att = pexp / jnp.sum(pexp, axis=1, keepdims=True)
    return _bdot(att, wh)


def _elu(x):
    return jnp.where(x > 0, x, jnp.exp(x) - 1.0)


def _kernel_b(pg_ref, q_ref, adj_ref, cpad_ref,
              g1_ref, b1_ref, w2T_ref, g2_ref, b2_ref, w3T_ref, g3_ref, b3_ref,
              c1T_ref, c1b_ref, bn1g_ref, bn1b_ref, c2T_ref, c2b_ref, bn2g_ref,
              bn2b_ref, gatW_ref, gata_ref, gatWo_ref, gatao_ref, c3T_ref,
              c3b_ref, out_ref, ctr_ref):
    pg = pg_ref[0].reshape(S, NSAMPLE, 128)
    q = q_ref[0].reshape(S, 1, 128)
    h = pg - q
    h = jnp.maximum(g1_ref[...] * INV_BN * h + b1_ref[...], 0.0)
    h = h.reshape(S * NSAMPLE, 128)
    h = _bdot(h, w2T_ref[...])
    h = jnp.maximum(g2_ref[...] * INV_BN * h + b2_ref[...], 0.0)
    h = _bdot(h, w3T_ref[...])
    h = jnp.maximum(g3_ref[...] * INV_BN * h + b3_ref[...], 0.0)
    feat = jnp.max(h.reshape(S, NSAMPLE, 128), axis=1)

    net = jnp.dot(feat, c1T_ref[...], precision=HIGH,
                  preferred_element_type=jnp.float32) + c1b_ref[...]
    net = jnp.maximum(bn1g_ref[...] * net * INV_BN + bn1b_ref[...], 0.0)
    net = jnp.dot(net, c2T_ref[...], precision=HIGH,
                  preferred_element_type=jnp.float32) + c2b_ref[...]
    net = jnp.maximum(bn2g_ref[...] * net * INV_BN + bn2b_ref[...], 0.0)

    adj = adj_ref[0]
    heads = []
    for i in range(NHEADS):
        heads.append(_elu(_gat_head(net, gatW_ref[i], gata_ref[i:i + 1], adj)))
    hcat = jnp.concatenate(heads, axis=1)
    gat_out = _elu(_gat_head(hcat, gatWo_ref[...], gatao_ref[...], adj))

    net3 = jnp.dot(gat_out, c3T_ref[...], precision=HIGH,
                   preferred_element_type=jnp.float32) + c3b_ref[...]
    out_ref[0] = net3
    center = cpad_ref[0][:, 0:3] + net3[:, 2:5]
    ctr_ref[0] = jnp.concatenate([center, jnp.zeros((S, 5), jnp.float32)], axis=1)


def _full(x):
    return pl.BlockSpec(x.shape, lambda b: (0,) * x.ndim)


def _batched(shape):
    nd = len(shape)
    return pl.BlockSpec((1,) + shape, lambda b, _nd=nd: (b,) + (0,) * _nd)


def kernel(xyz, features, sample_inds, sa_w1, sa_g1, sa_b1, sa_w2, sa_g2, sa_b2,
           sa_w3, sa_g3, sa_b3, c1_w, c1_b, bn1_g, bn1_b, c2_w, c2_b, bn2_g,
           bn2_b, gat_W, gat_a, gat_Wo, gat_ao, c3_w, c3_b):
    f32 = jnp.float32
    xyz = xyz.astype(f32)
    xyz_pad = jnp.concatenate([xyz, jnp.zeros((B, N, 13), f32)], axis=2)

    inds_flat = (sample_inds.astype(jnp.int32)
                 + (jnp.arange(B, dtype=jnp.int32) * N)[:, None]).reshape(-1)
    inds_pad = jnp.concatenate([inds_flat,
                                jnp.zeros((B * S,), jnp.int32)])
    xyz_pad128 = jnp.concatenate([xyz, jnp.zeros((B, N, 125), f32)], axis=2)
    cpad = _sc_gather_rows(xyz_pad128.reshape(B * N, 128), inds_pad, 128)
    cpad = cpad[:B * S, :16].reshape(B, S, 16)

    xyzT = jnp.swapaxes(xyz_pad[:, :, :8], 1, 2)
    cT = jnp.swapaxes(cpad[:, :, :8], 1, 2)
    featT = jnp.swapaxes(features.astype(f32), 1, 2)
    w1xT = jnp.concatenate([jnp.transpose(sa_w1[:, :3]).astype(f32),
                            jnp.zeros((13, 128), f32)], axis=0)
    w1fT = jnp.transpose(sa_w1[:, 3:]).astype(f32)

    gi, adjmask, p, q = pl.pallas_call(
        _kernel_a,
        grid=(B,),
        in_specs=[_batched((N, 16)), _batched((8, N)), _batched((N, C)),
                  _batched((S, 16)), _batched((8, S)), _full(w1xT), _full(w1fT)],
        out_specs=[_batched((S, NSAMPLE)), _batched((S, S)), _batched((N, 128)),
                   _batched((S, 128))],
        out_shape=[jax.ShapeDtypeStruct((B, S, NSAMPLE), jnp.int32),
                   jax.ShapeDtypeStruct((B, S, S), f32),
                   jax.ShapeDtypeStruct((B, N, 128), f32),
                   jax.ShapeDtypeStruct((B, S, 128), f32)],
    )(xyz_pad, xyzT, featT, cpad, cT, w1xT, w1fT)

    pg = _sc_gather_rows(p.reshape(B * N, 128), gi.reshape(-1), 128)
    pg = pg.reshape(B, S * NSAMPLE, 128)

    r1 = lambda a: a.astype(f32).reshape(1, -1)
    tr = lambda a: jnp.transpose(a.astype(f32))
    c3T = jnp.concatenate([tr(c3_w), jnp.zeros((128, 128 - OUT_CH), f32)], axis=1)
    c3b = jnp.concatenate([c3_b.astype(f32),
                           jnp.zeros((128 - OUT_CH,), f32)]).reshape(1, -1)

    net3, ctr = pl.pallas_call(
        _kernel_b,
        grid=(B,),
        in_specs=[_batched((S * NSAMPLE, 128)), _batched((S, 128)),
                  _batched((S, S)), _batched((S, 16)),
                  _full(r1(sa_g1)), _full(r1(sa_b1)), _full(tr(sa_w2)),
                  _full(r1(sa_g2)), _full(r1(sa_b2)), _full(tr(sa_w3)),
                  _full(r1(sa_g3)), _full(r1(sa_b3)), _full(tr(c1_w)),
                  _full(r1(c1_b)), _full(r1(bn1_g)), _full(r1(bn1_b)),
                  _full(tr(c2_w)), _full(r1(c2_b)), _full(r1(bn2_g)),
                  _full(r1(bn2_b)), _full(gat_W.astype(f32)),
                  _full(gat_a.astype(f32)), _full(gat_Wo.astype(f32)),
                  _full(gat_ao.astype(f32).reshape(1, -1)), _full(c3T),
                  _full(c3b)],
        out_specs=[_batched((S, 128)), _batched((S, 8))],
        out_shape=[jax.ShapeDtypeStruct((B, S, 128), f32),
                   jax.ShapeDtypeStruct((B, S, 8), f32)],
    )(pg, q, adjmask, cpad, r1(sa_g1), r1(sa_b1), tr(sa_w2), r1(sa_g2),
      r1(sa_b2), tr(sa_w3), r1(sa_g3), r1(sa_b3), tr(c1_w), r1(c1_b),
      r1(bn1_g), r1(bn1_b), tr(c2_w), r1(c2_b), r1(bn2_g), r1(bn2_b),
      gat_W.astype(f32), gat_a.astype(f32), gat_Wo.astype(f32),
      gat_ao.astype(f32).reshape(1, -1), c3T, c3b)

    return jnp.concatenate([net3[:, :, :OUT_CH], ctr[:, :, :3]], axis=2)


# speedup vs baseline: 10.9159x; 1.5484x over previous
import functools
import math

import jax
import jax.numpy as jnp
from jax.experimental import pallas as pl
from jax.experimental.pallas import tpu as pltpu
from jax.experimental.pallas import tpu_sc as plsc

B, N, C = 8, 1024, 256
S = 256
NSAMPLE = 16
RADIUS = 0.3
NHEADS = 4
NHID = 128
OUT_CH = 119
INV_BN = 1.0 / math.sqrt(1.0 + 1e-5)
HIGH = jax.lax.Precision.HIGHEST


def _sc_gather_rows(x, idx, window):
    n = idx.shape[0]
    V = x.shape[1]
    idx2 = idx.reshape(1, n)
    mesh = plsc.VectorSubcoreMesh(core_axis_name="core", subcore_axis_name="subcore")

    @pl.kernel(out_type=jax.ShapeDtypeStruct((n, V), x.dtype), mesh=mesh)
    def k(x_hbm, i_hbm, o_hbm):
        def body(i_vmem, o_vmem):
            pltpu.sync_copy(x_hbm.at[i_vmem.at[0]], o_vmem)

        pltpu.emit_pipeline(
            body,
            grid=(n // window,),
            in_specs=[pl.BlockSpec((1, window), lambda i: (0, i))],
            out_specs=[pl.BlockSpec((window, V), lambda i: (i, 0))],
            core_axis_name=("core", "subcore"),
            dimension_semantics=(pltpu.PARALLEL,),
        )(i_hbm, o_hbm)

    return k(x, idx2)


def _kernel_a(xyzp_ref, xyzT_ref, featT_ref, inds_ref, w1xT_ref, w1fT_ref,
              gi_ref, adj_ref, p_ref, q_ref, c_ref):
    b = pl.program_id(0)
    r2 = RADIUS * RADIUS
    inv_r = 1.0 / RADIUS

    xyzp = xyzp_ref[0]
    ic = inds_ref[0][:, 0:1]
    onehot = (jax.lax.broadcasted_iota(jnp.int32, (S, N), 1) == ic
              ).astype(jnp.bfloat16)
    hi = xyzp.astype(jnp.bfloat16)
    r_ = xyzp - hi.astype(jnp.float32)
    mid = r_.astype(jnp.bfloat16)
    lo = (r_ - mid.astype(jnp.float32)).astype(jnp.bfloat16)
    g_hi = jnp.dot(onehot, hi, preferred_element_type=jnp.float32)
    g_mid = jnp.dot(onehot, mid, preferred_element_type=jnp.float32)
    g_lo = jnp.dot(onehot, lo, preferred_element_type=jnp.float32)
    cpad = g_hi + (g_mid + g_lo)
    c_ref[0] = cpad

    featT = featT_ref[0]
    p = (jnp.dot(xyzp * inv_r, w1xT_ref[...], precision=HIGH,
                 preferred_element_type=jnp.float32)
         + jnp.dot(featT, w1fT_ref[...], precision=HIGH,
                   preferred_element_type=jnp.float32))
    p_ref[0] = p
    q_ref[0] = jnp.dot(cpad * inv_r, w1xT_ref[...], precision=HIGH,
                       preferred_element_type=jnp.float32)

    cx = cpad[:, 0:1]
    cy = cpad[:, 1:2]
    cz = cpad[:, 2:3]
    xx = xyzT_ref[0, 0:1, :]
    xy = xyzT_ref[0, 1:2, :]
    xz = xyzT_ref[0, 2:3, :]
    dx = cx - xx
    dy = cy - xy
    dz = cz - xz
    d2 = dx * dx + dy * dy + dz * dz
    iota_n = jax.lax.broadcasted_iota(jnp.int32, (S, N), 1).astype(jnp.float32)
    cand = jnp.where(d2 < r2, iota_n, float(N))
    cols = []
    for _ in range(NSAMPLE):
        m = jnp.min(cand, axis=1, keepdims=True)
        cols.append(m)
        cand = jnp.where(cand == m, float(N), cand)
    gi = jnp.concatenate(cols, axis=1)
    gi = jnp.where(gi == float(N), cols[0], gi)
    gi_ref[0] = gi.astype(jnp.int32) + b * N

    cT = jnp.transpose(cpad)
    ctx = cT[0:1, :]
    cty = cT[1:2, :]
    ctz = cT[2:3, :]
    ddx = cx - ctx
    ddy = cy - cty
    ddz = cz - ctz
    dist = jnp.sqrt(ddx * ddx + ddy * ddy + ddz * ddz)
    iota_s = jax.lax.broadcasted_iota(jnp.int32, (S, S), 1).astype(jnp.float32)
    amask = jnp.zeros((S, S), jnp.float32)
    for _ in range(NSAMPLE):
        m = jnp.min(dist, axis=1, keepdims=True)
        eq = dist == m
        jsel = jnp.min(jnp.where(eq, iota_s, 4096.0), axis=1, keepdims=True)
        one = iota_s == jsel
        amask = jnp.where(one, 1.0, amask)
        dist = jnp.where(one, 1e30, dist)
    amask = jnp.maximum(amask, jnp.transpose(amask))
    eye = (iota_s ==
           jax.lax.broadcasted_iota(jnp.int32, (S, S), 0).astype(jnp.float32))
    adj_ref[0] = jnp.where(eye, 1.0, amask)


def _bdot(a, b):
    return jnp.dot(a.astype(jnp.bfloat16), b.astype(jnp.bfloat16),
                   preferred_element_type=jnp.float32)


def _gat_head(x, W, a_row, adjmask):
    fp = W.shape[1]
    wh = jnp.dot(x, W, precision=HIGH, preferred_element_type=jnp.float32)
    u = jnp.sum(wh * a_row[:, :fp], axis=1, keepdims=True)
    v = jnp.sum(wh * a_row[:, fp:], axis=1, keepdims=True)
    vT = jnp.transpose(jnp.broadcast_to(v, (S, S)))
    e = u + vT
    e = jnp.where(e > 0, e, 0.2 * e)
    e = jnp.where(adjmask > 0, e, -9e15)
    m = jnp.max(e, axis=1, keepdims=True)
    pexp = jnp.exp(e - m)
    att = pexp / jnp.sum(pexp, axis=1, keepdims=True)
    return _bdot(att, wh)


def _elu(x):
    return jnp.where(x > 0, x, jnp.exp(x) - 1.0)


def _kernel_b(pg_ref, q_ref, adj_ref, cpad_ref,
              g1_ref, b1_ref, w2T_ref, g2_ref, b2_ref, w3T_ref, g3_ref, b3_ref,
              c1T_ref, c1b_ref, bn1g_ref, bn1b_ref, c2T_ref, c2b_ref, bn2g_ref,
              bn2b_ref, gatW_ref, gata_ref, gatWo_ref, gatao_ref, c3T_ref,
              c3b_ref, out_ref, ctr_ref):
    pg = pg_ref[0].reshape(S, NSAMPLE, 128)
    q = q_ref[0].reshape(S, 1, 128)
    h = pg - q
    h = jnp.maximum(g1_ref[...] * INV_BN * h + b1_ref[...], 0.0)
    h = h.reshape(S * NSAMPLE, 128)
    h = _bdot(h, w2T_ref[...])
    h = jnp.maximum(g2_ref[...] * INV_BN * h + b2_ref[...], 0.0)
    h = _bdot(h, w3T_ref[...])
    h = jnp.maximum(g3_ref[...] * INV_BN * h + b3_ref[...], 0.0)
    feat = jnp.max(h.reshape(S, NSAMPLE, 128), axis=1)

    net = jnp.dot(feat, c1T_ref[...], precision=HIGH,
                  preferred_element_type=jnp.float32) + c1b_ref[...]
    net = jnp.maximum(bn1g_ref[...] * net * INV_BN + bn1b_ref[...], 0.0)
    net = jnp.dot(net, c2T_ref[...], precision=HIGH,
                  preferred_element_type=jnp.float32) + c2b_ref[...]
    net = jnp.maximum(bn2g_ref[...] * net * INV_BN + bn2b_ref[...], 0.0)

    adj = adj_ref[0]
    heads = []
    for i in range(NHEADS):
        heads.append(_elu(_gat_head(net, gatW_ref[i], gata_ref[i:i + 1], adj)))
    hcat = jnp.concatenate(heads, axis=1)
    gat_out = _elu(_gat_head(hcat, gatWo_ref[...], gatao_ref[...], adj))

    net3 = jnp.dot(gat_out, c3T_ref[...], precision=HIGH,
                   preferred_element_type=jnp.float32) + c3b_ref[...]
    out_ref[0] = net3
    center = cpad_ref[0][:, 0:3] + net3[:, 2:5]
    ctr_ref[0] = jnp.concatenate([center, jnp.zeros((S, 5), jnp.float32)], axis=1)


def _full(x):
    return pl.BlockSpec(x.shape, lambda b: (0,) * x.ndim)


def _batched(shape):
    nd = len(shape)
    return pl.BlockSpec((1,) + shape, lambda b, _nd=nd: (b,) + (0,) * _nd)


def kernel(xyz, features, sample_inds, sa_w1, sa_g1, sa_b1, sa_w2, sa_g2, sa_b2,
           sa_w3, sa_g3, sa_b3, c1_w, c1_b, bn1_g, bn1_b, c2_w, c2_b, bn2_g,
           bn2_b, gat_W, gat_a, gat_Wo, gat_ao, c3_w, c3_b):
    f32 = jnp.float32
    xyz = xyz.astype(f32)
    xyz_pad = jnp.concatenate([xyz, jnp.zeros((B, N, 13), f32)], axis=2)
    inds_col = jnp.broadcast_to(sample_inds.astype(jnp.int32)[:, :, None],
                                (B, S, 8))
    xyzT = jnp.swapaxes(xyz_pad[:, :, :8], 1, 2)
    featT = jnp.swapaxes(features.astype(f32), 1, 2)
    w1xT = jnp.concatenate([jnp.transpose(sa_w1[:, :3]).astype(f32),
                            jnp.zeros((13, 128), f32)], axis=0)
    w1fT = jnp.transpose(sa_w1[:, 3:]).astype(f32)

    gi, adjmask, p, q, cpad = pl.pallas_call(
        _kernel_a,
        grid=(B,),
        in_specs=[_batched((N, 16)), _batched((8, N)), _batched((N, C)),
                  _batched((S, 8)), _full(w1xT), _full(w1fT)],
        out_specs=[_batched((S, NSAMPLE)), _batched((S, S)), _batched((N, 128)),
                   _batched((S, 128)), _batched((S, 16))],
        out_shape=[jax.ShapeDtypeStruct((B, S, NSAMPLE), jnp.int32),
                   jax.ShapeDtypeStruct((B, S, S), f32),
                   jax.ShapeDtypeStruct((B, N, 128), f32),
                   jax.ShapeDtypeStruct((B, S, 128), f32),
                   jax.ShapeDtypeStruct((B, S, 16), f32)],
    )(xyz_pad, xyzT, featT, inds_col, w1xT, w1fT)

    pg = _sc_gather_rows(p.reshape(B * N, 128), gi.reshape(-1), 128)
    pg = pg.reshape(B, S * NSAMPLE, 128)

    r1 = lambda a: a.astype(f32).reshape(1, -1)
    tr = lambda a: jnp.transpose(a.astype(f32))
    c3T = jnp.concatenate([tr(c3_w), jnp.zeros((128, 128 - OUT_CH), f32)], axis=1)
    c3b = jnp.concatenate([c3_b.astype(f32),
                           jnp.zeros((128 - OUT_CH,), f32)]).reshape(1, -1)

    net3, ctr = pl.pallas_call(
        _kernel_b,
        grid=(B,),
        in_specs=[_batched((S * NSAMPLE, 128)), _batched((S, 128)),
                  _batched((S, S)), _batched((S, 16)),
                  _full(r1(sa_g1)), _full(r1(sa_b1)), _full(tr(sa_w2)),
                  _full(r1(sa_g2)), _full(r1(sa_b2)), _full(tr(sa_w3)),
                  _full(r1(sa_g3)), _full(r1(sa_b3)), _full(tr(c1_w)),
                  _full(r1(c1_b)), _full(r1(bn1_g)), _full(r1(bn1_b)),
                  _full(tr(c2_w)), _full(r1(c2_b)), _full(r1(bn2_g)),
                  _full(r1(bn2_b)), _full(gat_W.astype(f32)),
                  _full(gat_a.astype(f32)), _full(gat_Wo.astype(f32)),
                  _full(gat_ao.astype(f32).reshape(1, -1)), _full(c3T),
                  _full(c3b)],
        out_specs=[_batched((S, 128)), _batched((S, 8))],
        out_shape=[jax.ShapeDtypeStruct((B, S, 128), f32),
                   jax.ShapeDtypeStruct((B, S, 8), f32)],
    )(pg, q, adjmask, cpad, r1(sa_g1), r1(sa_b1), tr(sa_w2), r1(sa_g2),
      r1(sa_b2), tr(sa_w3), r1(sa_g3), r1(sa_b3), tr(c1_w), r1(c1_b),
      r1(bn1_g), r1(bn1_b), tr(c2_w), r1(c2_b), r1(bn2_g), r1(bn2_b),
      gat_W.astype(f32), gat_a.astype(f32), gat_Wo.astype(f32),
      gat_ao.astype(f32).reshape(1, -1), c3T, c3b)

    return jnp.concatenate([net3[:, :, :OUT_CH], ctr[:, :, :3]], axis=2)
